# baseline (speedup 1.0000x reference)
import jax
import jax.numpy as jnp
from jax import lax
from jax.experimental import pallas as pl
from jax.experimental.pallas import tpu as pltpu
from jax.experimental.pallas import tpu_sc as plsc

N = 10000
NP = 10112
E = 320000
D = 128
DH = D // 2
BN_EPS = 1e-5

NC = 2
NS = 16
CHUNK = 128
FAST = 0
ROUND = 32
ROUNDS_F = 4
ROUNDS_S = 1
E_FAST = NS * ROUNDS_F * ROUND * CHUNK
E_SLOW = NS * ROUNDS_S * ROUND * CHUNK
E_PAD = E_FAST + E_SLOW
TRASH_ROW = N
ROWS_PER_TILE = NP // NS


def _dense_body(x_ref, g_ref, b_ref, wt_ref, h_ref, hs_ref):
    x = x_ref[...]
    mean = jnp.mean(x, axis=0, keepdims=True)
    xc = x - mean
    var = jnp.mean(xc * xc, axis=0, keepdims=True)
    inv = lax.rsqrt(var + BN_EPS)
    xh = jnp.maximum(xc * (inv * g_ref[...]) + b_ref[...], 0.0)
    h = jnp.dot(xh, wt_ref[...], preferred_element_type=jnp.float32)
    h_ref[...] = h
    hs_ref[0:N, :] = 0.5 * h * h


_dense_call = pl.pallas_call(
    _dense_body,
    out_shape=(
        jax.ShapeDtypeStruct((N, D), jnp.float32),
        jax.ShapeDtypeStruct((NP, D), jnp.float32),
    ),
)


def _sc_body(h_hbm, ef_hbm, es_hbm, hs_hbm, out_hbm, cidx, rows, acc, gsem):
    c = lax.axis_index("c")
    s = lax.axis_index("s")

    r0 = s * ROWS_PER_TILE
    pltpu.sync_copy(hs_hbm.at[pl.ds(r0, ROWS_PER_TILE)], acc.at[pl.ds(r0, ROWS_PER_TILE)])
    plsc.subcore_barrier()

    def chunk(i, carry):
        pltpu.async_copy(h_hbm.at[cidx.at[i, 0]], rows, gsem).wait()
        pltpu.sync_copy(rows, acc.at[cidx.at[i, 1]], add=True)
        return carry

    @pl.when(c == FAST)
    def _():
        def round_f(r, carry):
            pltpu.sync_copy(ef_hbm.at[s, pl.ds(r * ROUND, ROUND)], cidx)
            lax.fori_loop(0, ROUND, chunk, 0)
            return carry

        lax.fori_loop(0, ROUNDS_F, round_f, 0)

    @pl.when(c != FAST)
    def _():
        pltpu.sync_copy(es_hbm.at[s], cidx)
        lax.fori_loop(0, ROUND, chunk, 0)

    plsc.subcore_barrier()
    pltpu.sync_copy(
        acc.at[pl.ds(r0, ROWS_PER_TILE)],
        out_hbm.at[pl.ds(c * NP + r0, ROWS_PER_TILE)],
    )


_sc_call = pl.kernel(
    _sc_body,
    out_type=jax.ShapeDtypeStruct((2 * NP, D), jnp.float32),
    mesh=plsc.VectorSubcoreMesh(core_axis_name="c", subcore_axis_name="s"),
    scratch_types=[
        pltpu.VMEM((ROUND, 2, CHUNK), jnp.int32),
        pltpu.VMEM((CHUNK, D), jnp.float32),
        pltpu.VMEM_SHARED((NP, D), jnp.float32),
        pltpu.SemaphoreType.DMA,
    ],
)


def _combine_body(p_ref, o_ref):
    o_ref[...] = p_ref[0:N, :] + p_ref[NP : NP + N, :]


_combine_call = pl.pallas_call(
    _combine_body,
    out_shape=jax.ShapeDtypeStruct((N, D), jnp.float32),
)


def kernel(n_feat, edge_index, gamma, beta, W):
    edge_index = edge_index.astype(jnp.int32)
    h, hsh = _dense_call(
        n_feat,
        gamma.reshape(1, D),
        beta.reshape(1, D),
        W.T,
    )
    dst = edge_index[0]
    src = edge_index[1]
    pad = E_PAD - E
    src_p = jnp.concatenate([src, jnp.zeros((pad,), jnp.int32)])
    dst_p = jnp.concatenate([dst, jnp.full((pad,), TRASH_ROW, jnp.int32)])
    def blocks(sl, dl, n_chunks):
        s4 = sl.reshape(NS, n_chunks, 1, CHUNK)
        d4 = dl.reshape(NS, n_chunks, 1, CHUNK)
        return jnp.concatenate([s4, d4], axis=2)

    eidx_f = blocks(src_p[:E_FAST], dst_p[:E_FAST], ROUNDS_F * ROUND)
    eidx_s = blocks(src_p[E_FAST:], dst_p[E_FAST:], ROUNDS_S * ROUND)
    partials = _sc_call(h, eidx_f, eidx_s, hsh)
    return _combine_call(partials)


# speedup vs baseline: 1.2149x; 1.2149x over previous
import jax
import jax.numpy as jnp
from jax import lax
from jax.experimental import pallas as pl
from jax.experimental.pallas import tpu as pltpu
from jax.experimental.pallas import tpu_sc as plsc

N = 10000
NP = 10112
E = 320000
D = 128
BN_EPS = 1e-5

NC = 2
NS = 16
CHUNK = 128
CHUNKS_PER_TILE = 79
E_PAD = NC * NS * CHUNKS_PER_TILE * CHUNK
TRASH_ROW = N
ROWS_PER_TILE = NP // NS


def _dense_body(x_ref, g_ref, b_ref, wt_ref, h_ref, hs_ref):
    x = x_ref[...]
    mean = jnp.mean(x, axis=0, keepdims=True)
    xc = x - mean
    var = jnp.mean(xc * xc, axis=0, keepdims=True)
    inv = lax.rsqrt(var + BN_EPS)
    xh = jnp.maximum(xc * (inv * g_ref[...]) + b_ref[...], 0.0)
    h = jnp.dot(xh, wt_ref[...], preferred_element_type=jnp.float32)
    h_ref[...] = h
    hs_ref[0:N, :] = 0.5 * h * h


_dense_call = pl.pallas_call(
    _dense_body,
    out_shape=(
        jax.ShapeDtypeStruct((N, D), jnp.float32),
        jax.ShapeDtypeStruct((NP, D), jnp.float32),
    ),
)


def _sc_body(h_hbm, src_hbm, dst_hbm, hs_hbm, out_hbm, sidx, didx, rows, acc, gsem):
    c = lax.axis_index("c")
    s = lax.axis_index("s")

    r0 = s * ROWS_PER_TILE
    pltpu.sync_copy(hs_hbm.at[pl.ds(r0, ROWS_PER_TILE)], acc.at[pl.ds(r0, ROWS_PER_TILE)])
    pltpu.sync_copy(src_hbm.at[c, s], sidx)
    pltpu.sync_copy(dst_hbm.at[c, s], didx)
    plsc.subcore_barrier()

    def chunk(i, carry):
        pltpu.async_copy(h_hbm.at[sidx.at[i]], rows, gsem).wait()
        pltpu.sync_copy(rows, acc.at[didx.at[i]], add=True)
        return carry

    lax.fori_loop(0, CHUNKS_PER_TILE, chunk, 0)
    plsc.subcore_barrier()
    pltpu.sync_copy(
        acc.at[pl.ds(r0, ROWS_PER_TILE)],
        out_hbm.at[pl.ds(c * NP + r0, ROWS_PER_TILE)],
    )


_sc_call = pl.kernel(
    _sc_body,
    out_type=jax.ShapeDtypeStruct((2 * NP, D), jnp.float32),
    mesh=plsc.VectorSubcoreMesh(core_axis_name="c", subcore_axis_name="s"),
    scratch_types=[
        pltpu.VMEM((CHUNKS_PER_TILE, CHUNK), jnp.int32),
        pltpu.VMEM((CHUNKS_PER_TILE, CHUNK), jnp.int32),
        pltpu.VMEM((CHUNK, D), jnp.float32),
        pltpu.VMEM_SHARED((NP, D), jnp.float32),
        pltpu.SemaphoreType.DMA,
    ],
)


def _combine_body(p_ref, o_ref):
    o_ref[...] = p_ref[0:N, :] + p_ref[NP : NP + N, :]


_combine_call = pl.pallas_call(
    _combine_body,
    out_shape=jax.ShapeDtypeStruct((N, D), jnp.float32),
)


def kernel(n_feat, edge_index, gamma, beta, W):
    edge_index = edge_index.astype(jnp.int32)
    h, hsh = _dense_call(
        n_feat,
        gamma.reshape(1, D),
        beta.reshape(1, D),
        W.T,
    )
    dst = edge_index[0]
    src = edge_index[1]
    pad = E_PAD - E
    src_p = jnp.concatenate([src, jnp.zeros((pad,), jnp.int32)])
    dst_p = jnp.concatenate([dst, jnp.full((pad,), TRASH_ROW, jnp.int32)])
    src4 = src_p.reshape(NC, NS, CHUNKS_PER_TILE, CHUNK)
    dst4 = dst_p.reshape(NC, NS, CHUNKS_PER_TILE, CHUNK)
    partials = _sc_call(h, src4, dst4, hsh)
    return _combine_call(partials)
